# baseline (device time: 14260 ns/iter reference)
import jax
import jax.numpy as jnp
from jax import lax
from jax.experimental import pallas as pl
from jax.experimental.pallas import tpu as pltpu

N_DEV = 8
SCHEDULE = (1, 3, 4, 2, 5, 7, 6)


def kernel(x):
    m_per, n = x.shape

    def body(x_ref, out_ref, send_sems, recv_sems, ready_sems):
        my_pos = lax.axis_index("i")

        barrier_sem = pltpu.get_barrier_semaphore()
        pl.semaphore_signal(
            barrier_sem, inc=1,
            device_id=(my_pos,), device_id_type=pl.DeviceIdType.MESH,
        )
        pl.semaphore_wait(barrier_sem, 1)

        for i, mask in enumerate(SCHEDULE):
            pl.semaphore_signal(
                ready_sems.at[i], inc=1,
                device_id=(my_pos ^ mask,),
                device_id_type=pl.DeviceIdType.MESH,
            )

        out_ref[pl.ds(my_pos * m_per, m_per), :] = x_ref[:, :].astype(
            jnp.bfloat16
        )

        my_rows = out_ref.at[pl.ds(my_pos * m_per, m_per), :]
        sends = []
        for i, mask in enumerate(SCHEDULE):
            pl.semaphore_wait(ready_sems.at[i], 1)
            rdma = pltpu.make_async_remote_copy(
                src_ref=my_rows,
                dst_ref=my_rows,
                send_sem=send_sems.at[i],
                recv_sem=recv_sems.at[i],
                device_id=(my_pos ^ mask,),
                device_id_type=pl.DeviceIdType.MESH,
            )
            rdma.start()
            sends.append(rdma)

        for i, mask in enumerate(SCHEDULE):
            origin = my_pos ^ mask
            recv = pltpu.make_async_remote_copy(
                src_ref=my_rows,
                dst_ref=out_ref.at[pl.ds(origin * m_per, m_per), :],
                send_sem=send_sems.at[i],
                recv_sem=recv_sems.at[i],
                device_id=(origin,),
                device_id_type=pl.DeviceIdType.MESH,
            )
            recv.wait_recv()

        for rdma in sends:
            rdma.wait_send()

    return pl.pallas_call(
        body,
        out_shape=jax.ShapeDtypeStruct((N_DEV * m_per, n), jnp.bfloat16),
        in_specs=[pl.BlockSpec(memory_space=pltpu.VMEM)],
        out_specs=pl.BlockSpec(memory_space=pltpu.VMEM),
        scratch_shapes=[
            pltpu.SemaphoreType.DMA((N_DEV - 1,)),
            pltpu.SemaphoreType.DMA((N_DEV - 1,)),
            pltpu.SemaphoreType.REGULAR((N_DEV - 1,)),
        ],
        compiler_params=pltpu.CompilerParams(collective_id=0),
    )(x)


# device time: 13016 ns/iter; 1.0956x vs baseline; 1.0956x over previous
import jax
import jax.numpy as jnp
from jax import lax
from jax.experimental import pallas as pl
from jax.experimental.pallas import tpu as pltpu

N_DEV = 8
OFFSETS = (1, 2, 3, 4, 5, 6, 7)


def kernel(x):
    m_per, n = x.shape

    def body(x_ref, out_ref, send_sems, recv_sems, ready_sems):
        my_pos = lax.axis_index("i")

        for i, off in enumerate(OFFSETS):
            pl.semaphore_signal(
                ready_sems.at[i], inc=1,
                device_id=((my_pos - off) % N_DEV,),
                device_id_type=pl.DeviceIdType.MESH,
            )

        barrier_sem = pltpu.get_barrier_semaphore()
        pl.semaphore_signal(
            barrier_sem, inc=1,
            device_id=(my_pos,), device_id_type=pl.DeviceIdType.MESH,
        )
        pl.semaphore_wait(barrier_sem, 1)

        out_ref[pl.ds(my_pos * m_per, m_per), :] = x_ref[:, :].astype(
            jnp.bfloat16
        )

        sends = []
        my_rows = out_ref.at[pl.ds(my_pos * m_per, m_per), :]
        for i, off in enumerate(OFFSETS):
            pl.semaphore_wait(ready_sems.at[i], 1)
            rdma = pltpu.make_async_remote_copy(
                src_ref=my_rows,
                dst_ref=my_rows,
                send_sem=send_sems.at[i],
                recv_sem=recv_sems.at[i],
                device_id=((my_pos + off) % N_DEV,),
                device_id_type=pl.DeviceIdType.MESH,
            )
            rdma.start()
            sends.append(rdma)

        for i, off in enumerate(OFFSETS):
            origin = (my_pos - off) % N_DEV
            recv = pltpu.make_async_remote_copy(
                src_ref=my_rows,
                dst_ref=out_ref.at[pl.ds(origin * m_per, m_per), :],
                send_sem=send_sems.at[i],
                recv_sem=recv_sems.at[i],
                device_id=(origin,),
                device_id_type=pl.DeviceIdType.MESH,
            )
            recv.wait_recv()

        for rdma in sends:
            rdma.wait_send()

    return pl.pallas_call(
        body,
        out_shape=jax.ShapeDtypeStruct((N_DEV * m_per, n), jnp.bfloat16),
        in_specs=[pl.BlockSpec(memory_space=pltpu.VMEM)],
        out_specs=pl.BlockSpec(memory_space=pltpu.VMEM),
        scratch_shapes=[
            pltpu.SemaphoreType.DMA((N_DEV - 1,)),
            pltpu.SemaphoreType.DMA((N_DEV - 1,)),
            pltpu.SemaphoreType.REGULAR((N_DEV - 1,)),
        ],
        compiler_params=pltpu.CompilerParams(collective_id=0),
    )(x)
